# Initial kernel scaffold; baseline (speedup 1.0000x reference)
import jax
import jax.numpy as jnp
from jax.experimental import pallas as pl


def kernel(x, edge_index, adj_values, W1, a1, W2, ln_gamma, ln_beta):
    raise NotImplementedError("write your pallas kernel here")


# speedup vs baseline: 5.0172x; 5.0172x over previous
import functools

import jax
import jax.numpy as jnp
from jax import lax
from jax.experimental import pallas as pl
from jax.experimental.pallas import tpu as pltpu
from jax.experimental.pallas import tpu_sc as plsc

N = 10000
D = 128
E = 320000
NC = 2
NS = 16
NW = NC * NS
EW = E // NW
K = 80
NCHUNK = EW // K
NP = 10240
RPT = NP // NS
BLK = 512
GRID = (N + BLK - 1) // BLK


def _tc1_body(x_ref, w1_ref, at_ref, ab_ref, h1_ref, sr_ref, sc_ref):
    h1 = jnp.dot(x_ref[...], w1_ref[...], preferred_element_type=jnp.float32)
    h1_ref[...] = h1
    sr_ref[...] = jnp.sum(h1 * at_ref[...], axis=1).reshape(1, BLK)
    sc_ref[...] = jnp.sum(h1 * ab_ref[...], axis=1).reshape(1, BLK)


def _tc1(x, W1, a_top, a_bot):
    return pl.pallas_call(
        _tc1_body,
        grid=(GRID,),
        in_specs=[
            pl.BlockSpec((BLK, D), lambda i: (i, 0)),
            pl.BlockSpec((D, D), lambda i: (0, 0)),
            pl.BlockSpec((1, D), lambda i: (0, 0)),
            pl.BlockSpec((1, D), lambda i: (0, 0)),
        ],
        out_specs=[
            pl.BlockSpec((BLK, D), lambda i: (i, 0)),
            pl.BlockSpec((1, BLK), lambda i: (0, i)),
            pl.BlockSpec((1, BLK), lambda i: (0, i)),
        ],
        out_shape=[
            jax.ShapeDtypeStruct((N, D), jnp.float32),
            jax.ShapeDtypeStruct((1, N), jnp.float32),
            jax.ShapeDtypeStruct((1, N), jnp.float32),
        ],
    )(x, W1, a_top, a_bot)


def _tc2_body(acc_ref, w2_ref, h2_ref):
    h = jnp.maximum(acc_ref[0] + acc_ref[1], 0.0)
    h2_ref[...] = jnp.dot(h, w2_ref[...], preferred_element_type=jnp.float32)


def _tc2(acc, W2):
    return pl.pallas_call(
        _tc2_body,
        grid=(GRID,),
        in_specs=[
            pl.BlockSpec((NC, BLK, D), lambda i: (0, i, 0)),
            pl.BlockSpec((D, D), lambda i: (0, 0)),
        ],
        out_specs=pl.BlockSpec((BLK, D), lambda i: (i, 0)),
        out_shape=jax.ShapeDtypeStruct((N, D), jnp.float32),
    )(acc, W2)


def _tc3_body(acc_ref, x_ref, g_ref, b_ref, o_ref):
    h = jnp.maximum(acc_ref[0] + acc_ref[1], 0.0) + x_ref[...]
    mu = jnp.mean(h, axis=1, keepdims=True)
    d = h - mu
    var = jnp.mean(d * d, axis=1, keepdims=True)
    o_ref[...] = d * lax.rsqrt(var + 1e-5) * g_ref[...] + b_ref[...]


def _tc3(acc, x, gamma, beta):
    return pl.pallas_call(
        _tc3_body,
        grid=(GRID,),
        in_specs=[
            pl.BlockSpec((NC, BLK, D), lambda i: (0, i, 0)),
            pl.BlockSpec((BLK, D), lambda i: (i, 0)),
            pl.BlockSpec((1, D), lambda i: (0, 0)),
            pl.BlockSpec((1, D), lambda i: (0, 0)),
        ],
        out_specs=pl.BlockSpec((BLK, D), lambda i: (i, 0)),
        out_shape=jax.ShapeDtypeStruct((N, D), jnp.float32),
    )(acc, x, gamma, beta)


def _scale_rows(rows_v, w_v):
    @pl.loop(0, K // 16)
    def _(g):
        w16 = w_v[pl.ds(16 * g, 16)]
        for e in range(16):
            wb = jnp.broadcast_to(w16[e], (16,))
            j = 16 * g + e
            for q in range(D // 16):
                sl = pl.ds(16 * q, 16)
                rows_v[j, sl] = rows_v[j, sl] * wb


def _sc_prop1(h1, srow, scol, row, col, adj, zeros):
    mesh = plsc.VectorSubcoreMesh(core_axis_name="c", subcore_axis_name="s")

    @functools.partial(
        pl.kernel,
        out_type=[
            jax.ShapeDtypeStruct((NC, NP, D), jnp.float32),
            jax.ShapeDtypeStruct((E,), jnp.float32),
        ],
        mesh=mesh,
        compiler_params=pltpu.CompilerParams(needs_layout_passes=False),
        scratch_types=[
            pltpu.VMEM((N,), jnp.float32),
            pltpu.VMEM((N,), jnp.float32),
            pltpu.VMEM((K,), jnp.int32),
            pltpu.VMEM((K,), jnp.int32),
            pltpu.VMEM((K,), jnp.float32),
            pltpu.VMEM((K,), jnp.float32),
            pltpu.VMEM((K, D), jnp.float32),
            pltpu.VMEM_SHARED((NP, D), jnp.float32),
            pltpu.SemaphoreType.DMA,
        ],
    )
    def k(h1_hbm, srow_hbm, scol_hbm, row_hbm, col_hbm, adj_hbm, z_hbm,
          acc_hbm, vals_hbm,
          srow_v, scol_v, ir_v, ic_v, adj_v, w_v, rows_v, acc_sh, sem):
        cid = lax.axis_index("c")
        sid = lax.axis_index("s")
        wid = cid * NS + sid
        pltpu.sync_copy(z_hbm.at[pl.ds(sid * RPT, RPT)],
                        acc_sh.at[pl.ds(sid * RPT, RPT)])
        pltpu.sync_copy(srow_hbm, srow_v)
        pltpu.sync_copy(scol_hbm, scol_v)
        plsc.subcore_barrier()

        @pl.loop(0, NCHUNK)
        def _(c):
            base = wid * EW + c * K
            pltpu.sync_copy(row_hbm.at[pl.ds(base, K)], ir_v)
            pltpu.sync_copy(col_hbm.at[pl.ds(base, K)], ic_v)
            pltpu.sync_copy(adj_hbm.at[pl.ds(base, K)], adj_v)
            for j in range(K // 16):
                sl = pl.ds(16 * j, 16)
                s = (plsc.load_gather(srow_v, [ir_v[sl]])
                     + plsc.load_gather(scol_v, [ic_v[sl]]))
                s = jnp.where(s >= 0.0, s, 0.2 * s)
                w_v[sl] = adj_v[sl] / (1.0 + jnp.exp(-s))
            pltpu.sync_copy(w_v, vals_hbm.at[pl.ds(base, K)])
            pltpu.async_copy(h1_hbm.at[ic_v], rows_v, sem).wait()
            _scale_rows(rows_v, w_v)
            pltpu.sync_copy(rows_v, acc_sh.at[ir_v], add=True)

        plsc.subcore_barrier()
        pltpu.sync_copy(acc_sh.at[pl.ds(sid * RPT, RPT)],
                        acc_hbm.at[cid, pl.ds(sid * RPT, RPT)])

    return k(h1, srow, scol, row, col, adj, zeros)


def _sc_prop2(h2, row, col, vals, zeros):
    mesh = plsc.VectorSubcoreMesh(core_axis_name="c", subcore_axis_name="s")

    @functools.partial(
        pl.kernel,
        out_type=jax.ShapeDtypeStruct((NC, NP, D), jnp.float32),
        mesh=mesh,
        compiler_params=pltpu.CompilerParams(needs_layout_passes=False),
        scratch_types=[
            pltpu.VMEM((K,), jnp.int32),
            pltpu.VMEM((K,), jnp.int32),
            pltpu.VMEM((K,), jnp.float32),
            pltpu.VMEM((K, D), jnp.float32),
            pltpu.VMEM_SHARED((NP, D), jnp.float32),
            pltpu.SemaphoreType.DMA,
        ],
    )
    def k(h2_hbm, row_hbm, col_hbm, vals_hbm, z_hbm,
          acc_hbm,
          ir_v, ic_v, w_v, rows_v, acc_sh, sem):
        cid = lax.axis_index("c")
        sid = lax.axis_index("s")
        wid = cid * NS + sid
        pltpu.sync_copy(z_hbm.at[pl.ds(sid * RPT, RPT)],
                        acc_sh.at[pl.ds(sid * RPT, RPT)])
        plsc.subcore_barrier()

        @pl.loop(0, NCHUNK)
        def _(c):
            base = wid * EW + c * K
            pltpu.sync_copy(row_hbm.at[pl.ds(base, K)], ir_v)
            pltpu.sync_copy(col_hbm.at[pl.ds(base, K)], ic_v)
            pltpu.sync_copy(vals_hbm.at[pl.ds(base, K)], w_v)
            pltpu.async_copy(h2_hbm.at[ic_v], rows_v, sem).wait()
            _scale_rows(rows_v, w_v)
            pltpu.sync_copy(rows_v, acc_sh.at[ir_v], add=True)

        plsc.subcore_barrier()
        pltpu.sync_copy(acc_sh.at[pl.ds(sid * RPT, RPT)],
                        acc_hbm.at[cid, pl.ds(sid * RPT, RPT)])

    return k(h2, row, col, vals, zeros)


def kernel(x, edge_index, adj_values, W1, a1, W2, ln_gamma, ln_beta):
    row = edge_index[0].astype(jnp.int32)
    col = edge_index[1].astype(jnp.int32)
    a_top = jnp.reshape(a1[:D, 0], (1, D))
    a_bot = jnp.reshape(a1[D:, 0], (1, D))
    gamma = jnp.reshape(ln_gamma, (1, D))
    beta = jnp.reshape(ln_beta, (1, D))

    h1, sr, sc = _tc1(x, W1, a_top, a_bot)
    srow = jnp.reshape(sr, (N,))
    scol = jnp.reshape(sc, (N,))

    zeros = jnp.zeros((NP, D), jnp.float32)
    acc1, vals = _sc_prop1(h1, srow, scol, row, col, adj_values, zeros)
    h2 = _tc2(acc1, W2)
    acc2 = _sc_prop2(h2, row, col, vals, zeros)
    return _tc3(acc2, x, gamma, beta)


# speedup vs baseline: 9.7013x; 1.9336x over previous
import functools

import jax
import jax.numpy as jnp
from jax import lax
from jax.experimental import pallas as pl
from jax.experimental.pallas import tpu as pltpu
from jax.experimental.pallas import tpu_sc as plsc

N = 10000
D = 128
E = 320000
NC = 2
NS = 16
NW = NC * NS
EW = E // NW
K = 80
NCHUNK = EW // K
NP = 10240
RPT = NP // NS
BLK = 512
GRID = (N + BLK - 1) // BLK


def _tc1_body(x_ref, w1_ref, at_ref, ab_ref, h1_ref, sr_ref, sc_ref):
    h1 = jnp.dot(x_ref[...], w1_ref[...], preferred_element_type=jnp.float32)
    h1_ref[...] = h1
    sr_ref[...] = jnp.sum(h1 * at_ref[...], axis=1).reshape(1, BLK)
    sc_ref[...] = jnp.sum(h1 * ab_ref[...], axis=1).reshape(1, BLK)


def _tc1(x, W1, a_top, a_bot):
    return pl.pallas_call(
        _tc1_body,
        grid=(GRID,),
        in_specs=[
            pl.BlockSpec((BLK, D), lambda i: (i, 0)),
            pl.BlockSpec((D, D), lambda i: (0, 0)),
            pl.BlockSpec((1, D), lambda i: (0, 0)),
            pl.BlockSpec((1, D), lambda i: (0, 0)),
        ],
        out_specs=[
            pl.BlockSpec((BLK, D), lambda i: (i, 0)),
            pl.BlockSpec((1, BLK), lambda i: (0, i)),
            pl.BlockSpec((1, BLK), lambda i: (0, i)),
        ],
        out_shape=[
            jax.ShapeDtypeStruct((N, D), jnp.float32),
            jax.ShapeDtypeStruct((1, N), jnp.float32),
            jax.ShapeDtypeStruct((1, N), jnp.float32),
        ],
    )(x, W1, a_top, a_bot)


def _tc2_body(acc_ref, w2_ref, h2_ref):
    h = jnp.maximum(acc_ref[0] + acc_ref[1], 0.0)
    h2_ref[...] = jnp.dot(h, w2_ref[...], preferred_element_type=jnp.float32)


def _tc2(acc, W2):
    return pl.pallas_call(
        _tc2_body,
        grid=(GRID,),
        in_specs=[
            pl.BlockSpec((NC, BLK, D), lambda i: (0, i, 0)),
            pl.BlockSpec((D, D), lambda i: (0, 0)),
        ],
        out_specs=pl.BlockSpec((BLK, D), lambda i: (i, 0)),
        out_shape=jax.ShapeDtypeStruct((N, D), jnp.float32),
    )(acc, W2)


def _tc3_body(acc_ref, x_ref, g_ref, b_ref, o_ref):
    h = jnp.maximum(acc_ref[0] + acc_ref[1], 0.0) + x_ref[...]
    mu = jnp.mean(h, axis=1, keepdims=True)
    d = h - mu
    var = jnp.mean(d * d, axis=1, keepdims=True)
    o_ref[...] = d * lax.rsqrt(var + 1e-5) * g_ref[...] + b_ref[...]


def _tc3(acc, x, gamma, beta):
    return pl.pallas_call(
        _tc3_body,
        grid=(GRID,),
        in_specs=[
            pl.BlockSpec((NC, BLK, D), lambda i: (0, i, 0)),
            pl.BlockSpec((BLK, D), lambda i: (i, 0)),
            pl.BlockSpec((1, D), lambda i: (0, 0)),
            pl.BlockSpec((1, D), lambda i: (0, 0)),
        ],
        out_specs=pl.BlockSpec((BLK, D), lambda i: (i, 0)),
        out_shape=jax.ShapeDtypeStruct((N, D), jnp.float32),
    )(acc, x, gamma, beta)


NPAIR = (NCHUNK - 1) // 2

_SR, _SC, _W, _AJ = 0, 2, 4, 6


def _weights(small_v, buf, with_adj):
    for j in range(K // 16):
        sl = pl.ds(16 * j, 16)
        s = small_v[_SR + buf, sl] + small_v[_SC + buf, sl]
        s = jnp.where(s >= 0.0, s, 0.2 * s)
        w = 1.0 / (1.0 + jnp.exp(-s))
        if with_adj:
            w = w * small_v[_AJ + buf, sl]
        small_v[_W + buf, sl] = w


def _scale_rows(rows_v, small_v, wrow):
    @pl.loop(0, K // 16)
    def _(g):
        w16 = small_v[wrow, pl.ds(16 * g, 16)]
        for e in range(16):
            wb = jnp.broadcast_to(w16[e], (16,))
            j = 16 * g + e
            for q in range(D // 16):
                sl = pl.ds(16 * q, 16)
                rows_v[j, sl] = rows_v[j, sl] * wb


def _sc_prop1(h1, srow, scol, row, col3, adj, zeros):
    mesh = plsc.VectorSubcoreMesh(core_axis_name="c", subcore_axis_name="s")

    @functools.partial(
        pl.kernel,
        out_type=[
            jax.ShapeDtypeStruct((NC, NP, D), jnp.float32),
            jax.ShapeDtypeStruct((E,), jnp.float32),
        ],
        mesh=mesh,
        compiler_params=pltpu.CompilerParams(needs_layout_passes=False),
        scratch_types=[
            pltpu.VMEM((NCHUNK, K), jnp.int32),
            pltpu.VMEM((K,), jnp.int32),
            pltpu.VMEM((K,), jnp.int32),
            pltpu.VMEM((8, K), jnp.float32),
            pltpu.VMEM((K, D), jnp.float32),
            pltpu.VMEM((K, D), jnp.float32),
            pltpu.VMEM_SHARED((NP, D), jnp.float32),
            pltpu.SemaphoreType.DMA,
            pltpu.SemaphoreType.DMA,
            pltpu.SemaphoreType.DMA,
            pltpu.SemaphoreType.DMA,
            pltpu.SemaphoreType.DMA,
            pltpu.SemaphoreType.DMA,
            pltpu.SemaphoreType.DMA,
            pltpu.SemaphoreType.DMA,
        ],
    )
    def k(h1_hbm, srow_hbm, scol_hbm, row_hbm, col_hbm, adj_hbm, z_hbm,
          acc_hbm, vals_hbm,
          ic2_v, ir0_v, ir1_v, small_v, rows0_v, rows1_v, acc_sh,
          sem_i0, sem_i1, sem_t0, sem_t1, sem_g0, sem_g1, sem_v, sem_s):
        cid = lax.axis_index("c")
        sid = lax.axis_index("s")
        wid = cid * NS + sid
        pltpu.sync_copy(z_hbm.at[pl.ds(sid * RPT, RPT)],
                        acc_sh.at[pl.ds(sid * RPT, RPT)])
        pltpu.sync_copy(col_hbm.at[wid], ic2_v)
        plsc.subcore_barrier()

        rows = (rows0_v, rows1_v)
        irb = (ir0_v, ir1_v)
        sem_i = (sem_i0, sem_i1)
        sem_t = (sem_t0, sem_t1)
        sem_g = (sem_g0, sem_g1)

        def issue_a(c, buf):
            base = wid * EW + c * K
            hi = pltpu.async_copy(row_hbm.at[pl.ds(base, K)], irb[buf],
                                  sem_i[buf])
            hg = pltpu.async_copy(h1_hbm.at[ic2_v.at[c]], rows[buf],
                                  sem_g[buf])
            ha = pltpu.async_copy(adj_hbm.at[pl.ds(base, K)],
                                  small_v.at[_AJ + buf], sem_t[buf])
            return hi, hg, ha

        def issue_b(c, buf, hi):
            hi.wait()
            hsr = pltpu.async_copy(srow_hbm.at[irb[buf]],
                                   small_v.at[_SR + buf], sem_t[buf])
            hsc = pltpu.async_copy(scol_hbm.at[ic2_v.at[c]],
                                   small_v.at[_SC + buf], sem_t[buf])
            return hsr, hsc

        def finish(c, buf, hg, ha, hsr, hsc):
            base = wid * EW + c * K
            ha.wait(); hsr.wait(); hsc.wait()
            _weights(small_v, buf, True)
            pltpu.async_copy(small_v.at[_W + buf],
                             vals_hbm.at[pl.ds(base, K)], sem_v)
            hg.wait()
            _scale_rows(rows[buf], small_v, _W + buf)
            pltpu.async_copy(rows[buf], acc_sh.at[irb[buf]], sem_s, add=True)

        def drain(n_scatter, n_vals):
            for _ in range(n_scatter):
                pltpu.make_async_copy(h1_hbm.at[pl.ds(0, K)], rows0_v,
                                      sem_s).wait()
            for _ in range(n_vals):
                pltpu.make_async_copy(adj_hbm.at[pl.ds(0, K)],
                                      small_v.at[_W], sem_v).wait()

        @pl.loop(0, NPAIR)
        def _(t):
            a = 2 * t
            @pl.when(t > 0)
            def _():
                drain(2, 2)
            hia, hga, haa = issue_a(a, 0)
            hib, hgb, hab = issue_a(a + 1, 1)
            hsra, hsca = issue_b(a, 0, hia)
            hsrb, hscb = issue_b(a + 1, 1, hib)
            finish(a, 0, hga, haa, hsra, hsca)
            finish(a + 1, 1, hgb, hab, hsrb, hscb)

        drain(2, 2)
        tail = NCHUNK - 1
        hit, hgt, hat = issue_a(tail, 0)
        hsrt, hsct = issue_b(tail, 0, hit)
        finish(tail, 0, hgt, hat, hsrt, hsct)
        drain(1, 1)

        plsc.subcore_barrier()
        pltpu.sync_copy(acc_sh.at[pl.ds(sid * RPT, RPT)],
                        acc_hbm.at[cid, pl.ds(sid * RPT, RPT)])

    return k(h1, srow, scol, row, col3, adj, zeros)


def _sc_prop2(h2, row, col3, vals, zeros):
    mesh = plsc.VectorSubcoreMesh(core_axis_name="c", subcore_axis_name="s")

    @functools.partial(
        pl.kernel,
        out_type=jax.ShapeDtypeStruct((NC, NP, D), jnp.float32),
        mesh=mesh,
        compiler_params=pltpu.CompilerParams(needs_layout_passes=False),
        scratch_types=[
            pltpu.VMEM((NCHUNK, K), jnp.int32),
            pltpu.VMEM((K,), jnp.int32),
            pltpu.VMEM((K,), jnp.int32),
            pltpu.VMEM((8, K), jnp.float32),
            pltpu.VMEM((K, D), jnp.float32),
            pltpu.VMEM((K, D), jnp.float32),
            pltpu.VMEM_SHARED((NP, D), jnp.float32),
            pltpu.SemaphoreType.DMA,
            pltpu.SemaphoreType.DMA,
            pltpu.SemaphoreType.DMA,
            pltpu.SemaphoreType.DMA,
            pltpu.SemaphoreType.DMA,
            pltpu.SemaphoreType.DMA,
            pltpu.SemaphoreType.DMA,
        ],
    )
    def k(h2_hbm, row_hbm, col_hbm, vals_hbm, z_hbm,
          acc_hbm,
          ic2_v, ir0_v, ir1_v, small_v, rows0_v, rows1_v, acc_sh,
          sem_i0, sem_i1, sem_t0, sem_t1, sem_g0, sem_g1, sem_s):
        cid = lax.axis_index("c")
        sid = lax.axis_index("s")
        wid = cid * NS + sid
        pltpu.sync_copy(z_hbm.at[pl.ds(sid * RPT, RPT)],
                        acc_sh.at[pl.ds(sid * RPT, RPT)])
        pltpu.sync_copy(col_hbm.at[wid], ic2_v)
        plsc.subcore_barrier()

        rows = (rows0_v, rows1_v)
        irb = (ir0_v, ir1_v)
        sem_i = (sem_i0, sem_i1)
        sem_t = (sem_t0, sem_t1)
        sem_g = (sem_g0, sem_g1)

        def issue(c, buf):
            base = wid * EW + c * K
            hi = pltpu.async_copy(row_hbm.at[pl.ds(base, K)], irb[buf],
                                  sem_i[buf])
            hg = pltpu.async_copy(h2_hbm.at[ic2_v.at[c]], rows[buf],
                                  sem_g[buf])
            hw = pltpu.async_copy(vals_hbm.at[pl.ds(base, K)],
                                  small_v.at[_W + buf], sem_t[buf])
            return hi, hg, hw

        def finish(buf, hi, hg, hw):
            hw.wait()
            hg.wait()
            _scale_rows(rows[buf], small_v, _W + buf)
            hi.wait()
            pltpu.async_copy(rows[buf], acc_sh.at[irb[buf]], sem_s, add=True)

        def drain(n_scatter):
            for _ in range(n_scatter):
                pltpu.make_async_copy(h2_hbm.at[pl.ds(0, K)], rows0_v,
                                      sem_s).wait()

        @pl.loop(0, NPAIR)
        def _(t):
            a = 2 * t
            @pl.when(t > 0)
            def _():
                drain(2)
            hia, hga, hwa = issue(a, 0)
            hib, hgb, hwb = issue(a + 1, 1)
            finish(0, hia, hga, hwa)
            finish(1, hib, hgb, hwb)

        drain(2)
        tail = NCHUNK - 1
        hit, hgt, hwt = issue(tail, 0)
        finish(0, hit, hgt, hwt)
        drain(1)

        plsc.subcore_barrier()
        pltpu.sync_copy(acc_sh.at[pl.ds(sid * RPT, RPT)],
                        acc_hbm.at[cid, pl.ds(sid * RPT, RPT)])

    return k(h2, row, col3, vals, zeros)


def kernel(x, edge_index, adj_values, W1, a1, W2, ln_gamma, ln_beta):
    row = edge_index[0].astype(jnp.int32)
    col3 = edge_index[1].astype(jnp.int32).reshape(NW, NCHUNK, K)
    a_top = jnp.reshape(a1[:D, 0], (1, D))
    a_bot = jnp.reshape(a1[D:, 0], (1, D))
    gamma = jnp.reshape(ln_gamma, (1, D))
    beta = jnp.reshape(ln_beta, (1, D))

    h1, sr, sc = _tc1(x, W1, a_top, a_bot)
    srow = jnp.reshape(sr, (N,))
    scol = jnp.reshape(sc, (N,))

    zeros = jnp.zeros((NP, D), jnp.float32)
    acc1, vals = _sc_prop1(h1, srow, scol, row, col3, adj_values, zeros)
    h2 = _tc2(acc1, W2)
    acc2 = _sc_prop2(h2, row, col3, vals, zeros)
    return _tc3(acc2, x, gamma, beta)


# speedup vs baseline: 9.8462x; 1.0149x over previous
import functools

import jax
import jax.numpy as jnp
from jax import lax
from jax.experimental import pallas as pl
from jax.experimental.pallas import tpu as pltpu
from jax.experimental.pallas import tpu_sc as plsc

N = 10000
D = 128
E = 320000
NC = 2
NS = 16
NW = NC * NS
EW = E // NW
K = 80
NCHUNK = EW // K
NP = 10240
RPT = NP // NS
BLK = 512
GRID = (N + BLK - 1) // BLK


def _tc1_body(x_ref, w1_ref, at_ref, ab_ref, h1_ref, sr_ref, sc_ref):
    h1 = jnp.dot(x_ref[...], w1_ref[...], preferred_element_type=jnp.float32)
    h1_ref[...] = h1
    sr_ref[...] = jnp.sum(h1 * at_ref[...], axis=1).reshape(1, BLK)
    sc_ref[...] = jnp.sum(h1 * ab_ref[...], axis=1).reshape(1, BLK)


def _tc1(x, W1, a_top, a_bot):
    return pl.pallas_call(
        _tc1_body,
        grid=(GRID,),
        in_specs=[
            pl.BlockSpec((BLK, D), lambda i: (i, 0)),
            pl.BlockSpec((D, D), lambda i: (0, 0)),
            pl.BlockSpec((1, D), lambda i: (0, 0)),
            pl.BlockSpec((1, D), lambda i: (0, 0)),
        ],
        out_specs=[
            pl.BlockSpec((BLK, D), lambda i: (i, 0)),
            pl.BlockSpec((1, BLK), lambda i: (0, i)),
            pl.BlockSpec((1, BLK), lambda i: (0, i)),
        ],
        out_shape=[
            jax.ShapeDtypeStruct((N, D), jnp.float32),
            jax.ShapeDtypeStruct((1, N), jnp.float32),
            jax.ShapeDtypeStruct((1, N), jnp.float32),
        ],
    )(x, W1, a_top, a_bot)


def _tc2_body(acc_ref, w2_ref, h2_ref):
    h = jnp.maximum(acc_ref[0] + acc_ref[1], 0.0)
    h2_ref[...] = jnp.dot(h, w2_ref[...], preferred_element_type=jnp.float32)


def _tc2(acc, W2):
    return pl.pallas_call(
        _tc2_body,
        grid=(GRID,),
        in_specs=[
            pl.BlockSpec((NC, BLK, D), lambda i: (0, i, 0)),
            pl.BlockSpec((D, D), lambda i: (0, 0)),
        ],
        out_specs=pl.BlockSpec((BLK, D), lambda i: (i, 0)),
        out_shape=jax.ShapeDtypeStruct((N, D), jnp.float32),
    )(acc, W2)


def _tc3_body(acc_ref, x_ref, g_ref, b_ref, o_ref):
    h = jnp.maximum(acc_ref[0] + acc_ref[1], 0.0) + x_ref[...]
    mu = jnp.mean(h, axis=1, keepdims=True)
    d = h - mu
    var = jnp.mean(d * d, axis=1, keepdims=True)
    o_ref[...] = d * lax.rsqrt(var + 1e-5) * g_ref[...] + b_ref[...]


def _tc3(acc, x, gamma, beta):
    return pl.pallas_call(
        _tc3_body,
        grid=(GRID,),
        in_specs=[
            pl.BlockSpec((NC, BLK, D), lambda i: (0, i, 0)),
            pl.BlockSpec((BLK, D), lambda i: (i, 0)),
            pl.BlockSpec((1, D), lambda i: (0, 0)),
            pl.BlockSpec((1, D), lambda i: (0, 0)),
        ],
        out_specs=pl.BlockSpec((BLK, D), lambda i: (i, 0)),
        out_shape=jax.ShapeDtypeStruct((N, D), jnp.float32),
    )(acc, x, gamma, beta)


NPAIR = (NCHUNK - 1) // 2

_SR, _SC, _W, _AJ = 0, 2, 4, 6


def _unpack(pk2_v, c, ir_v, ic_v):
    for j in range(K // 16):
        sl = pl.ds(16 * j, 16)
        p = pk2_v[c, sl]
        ir_v[sl] = jax.lax.shift_right_logical(p, 14)
        ic_v[sl] = jax.lax.bitwise_and(p, 16383)


def _weights(small_v, buf):
    for j in range(K // 16):
        sl = pl.ds(16 * j, 16)
        s = small_v[_SR + buf, sl] + small_v[_SC + buf, sl]
        s = jnp.where(s >= 0.0, s, 0.2 * s)
        small_v[_W + buf, sl] = small_v[_AJ + buf, sl] / (1.0 + jnp.exp(-s))


def _scale_rows(rows_v, small_v, wrow):
    @pl.loop(0, K // 16)
    def _(g):
        w16 = small_v[wrow, pl.ds(16 * g, 16)]
        for e in range(16):
            wb = jnp.broadcast_to(w16[e], (16,))
            j = 16 * g + e
            for q in range(D // 16):
                sl = pl.ds(16 * q, 16)
                rows_v[j, sl] = rows_v[j, sl] * wb


def _sc_prop1(h1, srow, scol, pk3, adj, zeros):
    mesh = plsc.VectorSubcoreMesh(core_axis_name="c", subcore_axis_name="s")

    @functools.partial(
        pl.kernel,
        out_type=[
            jax.ShapeDtypeStruct((NC, NP, D), jnp.float32),
            jax.ShapeDtypeStruct((E,), jnp.float32),
        ],
        mesh=mesh,
        compiler_params=pltpu.CompilerParams(needs_layout_passes=False),
        scratch_types=[
            pltpu.VMEM((NCHUNK, K), jnp.int32),
            pltpu.VMEM((K,), jnp.int32),
            pltpu.VMEM((K,), jnp.int32),
            pltpu.VMEM((K,), jnp.int32),
            pltpu.VMEM((K,), jnp.int32),
            pltpu.VMEM((8, K), jnp.float32),
            pltpu.VMEM((K, D), jnp.float32),
            pltpu.VMEM((K, D), jnp.float32),
            pltpu.VMEM_SHARED((NP, D), jnp.float32),
            pltpu.SemaphoreType.DMA,
            pltpu.SemaphoreType.DMA,
            pltpu.SemaphoreType.DMA,
            pltpu.SemaphoreType.DMA,
            pltpu.SemaphoreType.DMA,
            pltpu.SemaphoreType.DMA,
        ],
    )
    def k(h1_hbm, srow_hbm, scol_hbm, pk_hbm, adj_hbm, z_hbm,
          acc_hbm, vals_hbm,
          pk2_v, ir0_v, ir1_v, ic0_v, ic1_v, small_v, rows0_v, rows1_v,
          acc_sh, sem_t0, sem_t1, sem_g0, sem_g1, sem_v, sem_s):
        cid = lax.axis_index("c")
        sid = lax.axis_index("s")
        wid = cid * NS + sid
        pltpu.sync_copy(z_hbm.at[pl.ds(sid * RPT, RPT)],
                        acc_sh.at[pl.ds(sid * RPT, RPT)])
        pltpu.sync_copy(pk_hbm.at[wid], pk2_v)
        plsc.subcore_barrier()

        rows = (rows0_v, rows1_v)
        irb = (ir0_v, ir1_v)
        icb = (ic0_v, ic1_v)
        sem_t = (sem_t0, sem_t1)
        sem_g = (sem_g0, sem_g1)

        def issue(c, buf):
            base = wid * EW + c * K
            _unpack(pk2_v, c, irb[buf], icb[buf])
            hg = pltpu.async_copy(h1_hbm.at[icb[buf]], rows[buf], sem_g[buf])
            hsr = pltpu.async_copy(srow_hbm.at[irb[buf]],
                                   small_v.at[_SR + buf], sem_t[buf])
            hsc = pltpu.async_copy(scol_hbm.at[icb[buf]],
                                   small_v.at[_SC + buf], sem_t[buf])
            ha = pltpu.async_copy(adj_hbm.at[pl.ds(base, K)],
                                  small_v.at[_AJ + buf], sem_t[buf])
            return hg, hsr, hsc, ha

        def finish(c, buf, hg, hsr, hsc, ha):
            base = wid * EW + c * K
            hsr.wait(); hsc.wait(); ha.wait()
            _weights(small_v, buf)
            pltpu.async_copy(small_v.at[_W + buf],
                             vals_hbm.at[pl.ds(base, K)], sem_v)
            hg.wait()
            _scale_rows(rows[buf], small_v, _W + buf)
            pltpu.async_copy(rows[buf], acc_sh.at[irb[buf]], sem_s, add=True)

        def drain(n_scatter, n_vals):
            for _ in range(n_scatter):
                pltpu.make_async_copy(h1_hbm.at[pl.ds(0, K)], rows0_v,
                                      sem_s).wait()
            for _ in range(n_vals):
                pltpu.make_async_copy(adj_hbm.at[pl.ds(0, K)],
                                      small_v.at[_W], sem_v).wait()

        @pl.loop(0, NPAIR)
        def _(t):
            a = 2 * t
            @pl.when(t > 0)
            def _():
                drain(2, 2)
            ha_ = issue(a, 0)
            hb_ = issue(a + 1, 1)
            finish(a, 0, *ha_)
            finish(a + 1, 1, *hb_)

        drain(2, 2)
        tail = NCHUNK - 1
        ht_ = issue(tail, 0)
        finish(tail, 0, *ht_)
        drain(1, 1)

        plsc.subcore_barrier()
        pltpu.sync_copy(acc_sh.at[pl.ds(sid * RPT, RPT)],
                        acc_hbm.at[cid, pl.ds(sid * RPT, RPT)])

    return k(h1, srow, scol, pk3, adj, zeros)


def _sc_prop2(h2, pk3, vals, zeros):
    mesh = plsc.VectorSubcoreMesh(core_axis_name="c", subcore_axis_name="s")

    @functools.partial(
        pl.kernel,
        out_type=jax.ShapeDtypeStruct((NC, NP, D), jnp.float32),
        mesh=mesh,
        compiler_params=pltpu.CompilerParams(needs_layout_passes=False),
        scratch_types=[
            pltpu.VMEM((NCHUNK, K), jnp.int32),
            pltpu.VMEM((K,), jnp.int32),
            pltpu.VMEM((K,), jnp.int32),
            pltpu.VMEM((K,), jnp.int32),
            pltpu.VMEM((K,), jnp.int32),
            pltpu.VMEM((8, K), jnp.float32),
            pltpu.VMEM((K, D), jnp.float32),
            pltpu.VMEM((K, D), jnp.float32),
            pltpu.VMEM_SHARED((NP, D), jnp.float32),
            pltpu.SemaphoreType.DMA,
            pltpu.SemaphoreType.DMA,
            pltpu.SemaphoreType.DMA,
            pltpu.SemaphoreType.DMA,
            pltpu.SemaphoreType.DMA,
        ],
    )
    def k(h2_hbm, pk_hbm, vals_hbm, z_hbm,
          acc_hbm,
          pk2_v, ir0_v, ir1_v, ic0_v, ic1_v, small_v, rows0_v, rows1_v,
          acc_sh, sem_t0, sem_t1, sem_g0, sem_g1, sem_s):
        cid = lax.axis_index("c")
        sid = lax.axis_index("s")
        wid = cid * NS + sid
        pltpu.sync_copy(z_hbm.at[pl.ds(sid * RPT, RPT)],
                        acc_sh.at[pl.ds(sid * RPT, RPT)])
        pltpu.sync_copy(pk_hbm.at[wid], pk2_v)
        plsc.subcore_barrier()

        rows = (rows0_v, rows1_v)
        irb = (ir0_v, ir1_v)
        icb = (ic0_v, ic1_v)
        sem_t = (sem_t0, sem_t1)
        sem_g = (sem_g0, sem_g1)

        def issue(c, buf):
            base = wid * EW + c * K
            _unpack(pk2_v, c, irb[buf], icb[buf])
            hg = pltpu.async_copy(h2_hbm.at[icb[buf]], rows[buf], sem_g[buf])
            hw = pltpu.async_copy(vals_hbm.at[pl.ds(base, K)],
                                  small_v.at[_W + buf], sem_t[buf])
            return hg, hw

        def finish(buf, hg, hw):
            hw.wait()
            hg.wait()
            _scale_rows(rows[buf], small_v, _W + buf)
            pltpu.async_copy(rows[buf], acc_sh.at[irb[buf]], sem_s, add=True)

        def drain(n_scatter):
            for _ in range(n_scatter):
                pltpu.make_async_copy(h2_hbm.at[pl.ds(0, K)], rows0_v,
                                      sem_s).wait()

        @pl.loop(0, NPAIR)
        def _(t):
            a = 2 * t
            @pl.when(t > 0)
            def _():
                drain(2)
            hga, hwa = issue(a, 0)
            hgb, hwb = issue(a + 1, 1)
            finish(0, hga, hwa)
            finish(1, hgb, hwb)

        drain(2)
        tail = NCHUNK - 1
        hgt, hwt = issue(tail, 0)
        finish(0, hgt, hwt)
        drain(1)

        plsc.subcore_barrier()
        pltpu.sync_copy(acc_sh.at[pl.ds(sid * RPT, RPT)],
                        acc_hbm.at[cid, pl.ds(sid * RPT, RPT)])

    return k(h2, pk3, vals, zeros)


def kernel(x, edge_index, adj_values, W1, a1, W2, ln_gamma, ln_beta):
    row = edge_index[0].astype(jnp.int32)
    col = edge_index[1].astype(jnp.int32)
    pk3 = (row * 16384 + col).reshape(NW, NCHUNK, K)
    a_top = jnp.reshape(a1[:D, 0], (1, D))
    a_bot = jnp.reshape(a1[D:, 0], (1, D))
    gamma = jnp.reshape(ln_gamma, (1, D))
    beta = jnp.reshape(ln_beta, (1, D))

    h1, sr, sc = _tc1(x, W1, a_top, a_bot)
    srow = jnp.reshape(sr, (N,))
    scol = jnp.reshape(sc, (N,))

    zeros = jnp.zeros((NP, D), jnp.float32)
    acc1, vals = _sc_prop1(h1, srow, scol, pk3, adj_values, zeros)
    h2 = _tc2(acc1, W2)
    acc2 = _sc_prop2(h2, pk3, vals, zeros)
    return _tc3(acc2, x, gamma, beta)


# speedup vs baseline: 9.9564x; 1.0112x over previous
import functools

import jax
import jax.numpy as jnp
from jax import lax
from jax.experimental import pallas as pl
from jax.experimental.pallas import tpu as pltpu
from jax.experimental.pallas import tpu_sc as plsc

N = 10000
D = 128
E = 320000
NC = 2
NS = 16
NW = NC * NS
EW = E // NW
K = 80
NCHUNK = EW // K
NP = 10240
RPT = NP // NS
BLK = 512
GRID = (N + BLK - 1) // BLK


def _tc1_body(x_ref, w1_ref, at_ref, ab_ref, h1_ref, sr_ref, sc_ref):
    h1 = jnp.dot(x_ref[...], w1_ref[...], preferred_element_type=jnp.float32)
    h1_ref[...] = h1
    sr_ref[...] = jnp.sum(h1 * at_ref[...], axis=1).reshape(1, BLK)
    sc_ref[...] = jnp.sum(h1 * ab_ref[...], axis=1).reshape(1, BLK)


def _tc1(x, W1, a_top, a_bot):
    return pl.pallas_call(
        _tc1_body,
        grid=(GRID,),
        in_specs=[
            pl.BlockSpec((BLK, D), lambda i: (i, 0)),
            pl.BlockSpec((D, D), lambda i: (0, 0)),
            pl.BlockSpec((1, D), lambda i: (0, 0)),
            pl.BlockSpec((1, D), lambda i: (0, 0)),
        ],
        out_specs=[
            pl.BlockSpec((BLK, D), lambda i: (i, 0)),
            pl.BlockSpec((1, BLK), lambda i: (0, i)),
            pl.BlockSpec((1, BLK), lambda i: (0, i)),
        ],
        out_shape=[
            jax.ShapeDtypeStruct((N, D), jnp.float32),
            jax.ShapeDtypeStruct((1, N), jnp.float32),
            jax.ShapeDtypeStruct((1, N), jnp.float32),
        ],
    )(x, W1, a_top, a_bot)


def _tc2_body(acc_ref, w2_ref, h2_ref):
    h = jnp.maximum(acc_ref[0] + acc_ref[1], 0.0)
    h2_ref[...] = jnp.dot(h, w2_ref[...], preferred_element_type=jnp.float32)


def _tc2(acc, W2):
    return pl.pallas_call(
        _tc2_body,
        grid=(GRID,),
        in_specs=[
            pl.BlockSpec((NC, BLK, D), lambda i: (0, i, 0)),
            pl.BlockSpec((D, D), lambda i: (0, 0)),
        ],
        out_specs=pl.BlockSpec((BLK, D), lambda i: (i, 0)),
        out_shape=jax.ShapeDtypeStruct((N, D), jnp.float32),
    )(acc, W2)


def _tc3_body(acc_ref, x_ref, g_ref, b_ref, o_ref):
    h = jnp.maximum(acc_ref[0] + acc_ref[1], 0.0) + x_ref[...]
    mu = jnp.mean(h, axis=1, keepdims=True)
    d = h - mu
    var = jnp.mean(d * d, axis=1, keepdims=True)
    o_ref[...] = d * lax.rsqrt(var + 1e-5) * g_ref[...] + b_ref[...]


def _tc3(acc, x, gamma, beta):
    return pl.pallas_call(
        _tc3_body,
        grid=(GRID,),
        in_specs=[
            pl.BlockSpec((NC, BLK, D), lambda i: (0, i, 0)),
            pl.BlockSpec((BLK, D), lambda i: (i, 0)),
            pl.BlockSpec((1, D), lambda i: (0, 0)),
            pl.BlockSpec((1, D), lambda i: (0, 0)),
        ],
        out_specs=pl.BlockSpec((BLK, D), lambda i: (i, 0)),
        out_shape=jax.ShapeDtypeStruct((N, D), jnp.float32),
    )(acc, x, gamma, beta)


NPAIR = (NCHUNK - 1) // 2

_SR, _SC, _W, _AJ = 0, 2, 4, 6


def _unpack(pk2_v, c, ir_v, ic_v):
    for j in range(K // 16):
        sl = pl.ds(16 * j, 16)
        p = pk2_v[c, sl]
        ir_v[sl] = jax.lax.shift_right_logical(p, 14)
        ic_v[sl] = jax.lax.bitwise_and(p, 16383)


def _weights(spack_v, ir_v, ic_v, small_v, buf):
    for j in range(K // 16):
        sl = pl.ds(16 * j, 16)
        g1 = plsc.load_gather(spack_v, [ir_v[sl]])
        g2 = plsc.load_gather(spack_v, [ic_v[sl]])
        sr = plsc.bitcast(jnp.bitwise_and(g1, jnp.int32(-65536)), jnp.float32)
        sc = plsc.bitcast(jnp.left_shift(g2, 16), jnp.float32)
        s = sr + sc
        s = jnp.where(s >= 0.0, s, 0.2 * s)
        small_v[_W + buf, sl] = small_v[_AJ + buf, sl] / (1.0 + jnp.exp(-s))


def _scale_rows(rows_v, small_v, wrow):
    @pl.loop(0, K // 16)
    def _(g):
        w16 = small_v[wrow, pl.ds(16 * g, 16)]
        for e in range(16):
            wb = jnp.broadcast_to(w16[e], (16,))
            j = 16 * g + e
            for q in range(D // 16):
                sl = pl.ds(16 * q, 16)
                rows_v[j, sl] = rows_v[j, sl] * wb


def _sc_prop1(h1, spack, pk3, adj, zeros):
    mesh = plsc.VectorSubcoreMesh(core_axis_name="c", subcore_axis_name="s")

    @functools.partial(
        pl.kernel,
        out_type=[
            jax.ShapeDtypeStruct((NC, NP, D), jnp.float32),
            jax.ShapeDtypeStruct((E,), jnp.float32),
        ],
        mesh=mesh,
        compiler_params=pltpu.CompilerParams(needs_layout_passes=False),
        scratch_types=[
            pltpu.VMEM((NCHUNK, K), jnp.int32),
            pltpu.VMEM((N,), jnp.int32),
            pltpu.VMEM((K,), jnp.int32),
            pltpu.VMEM((K,), jnp.int32),
            pltpu.VMEM((K,), jnp.int32),
            pltpu.VMEM((K,), jnp.int32),
            pltpu.VMEM((8, K), jnp.float32),
            pltpu.VMEM((K, D), jnp.float32),
            pltpu.VMEM((K, D), jnp.float32),
            pltpu.VMEM_SHARED((NP, D), jnp.float32),
            pltpu.SemaphoreType.DMA,
            pltpu.SemaphoreType.DMA,
            pltpu.SemaphoreType.DMA,
            pltpu.SemaphoreType.DMA,
            pltpu.SemaphoreType.DMA,
            pltpu.SemaphoreType.DMA,
        ],
    )
    def k(h1_hbm, spack_hbm, pk_hbm, adj_hbm, z_hbm,
          acc_hbm, vals_hbm,
          pk2_v, spack_v, ir0_v, ir1_v, ic0_v, ic1_v, small_v,
          rows0_v, rows1_v,
          acc_sh, sem_t0, sem_t1, sem_g0, sem_g1, sem_v, sem_s):
        cid = lax.axis_index("c")
        sid = lax.axis_index("s")
        wid = cid * NS + sid
        pltpu.sync_copy(z_hbm.at[pl.ds(sid * RPT, RPT)],
                        acc_sh.at[pl.ds(sid * RPT, RPT)])
        pltpu.sync_copy(pk_hbm.at[wid], pk2_v)
        pltpu.sync_copy(spack_hbm, spack_v)
        plsc.subcore_barrier()

        rows = (rows0_v, rows1_v)
        irb = (ir0_v, ir1_v)
        icb = (ic0_v, ic1_v)
        sem_t = (sem_t0, sem_t1)
        sem_g = (sem_g0, sem_g1)

        def issue(c, buf):
            base = wid * EW + c * K
            _unpack(pk2_v, c, irb[buf], icb[buf])
            hg = pltpu.async_copy(h1_hbm.at[icb[buf]], rows[buf], sem_g[buf])
            ha = pltpu.async_copy(adj_hbm.at[pl.ds(base, K)],
                                  small_v.at[_AJ + buf], sem_t[buf])
            return hg, ha

        def finish(c, buf, hg, ha):
            base = wid * EW + c * K
            ha.wait()
            _weights(spack_v, irb[buf], icb[buf], small_v, buf)
            pltpu.async_copy(small_v.at[_W + buf],
                             vals_hbm.at[pl.ds(base, K)], sem_v)
            hg.wait()
            _scale_rows(rows[buf], small_v, _W + buf)
            pltpu.async_copy(rows[buf], acc_sh.at[irb[buf]], sem_s, add=True)

        def drain(n_scatter, n_vals):
            for _ in range(n_scatter):
                pltpu.make_async_copy(h1_hbm.at[pl.ds(0, K)], rows0_v,
                                      sem_s).wait()
            for _ in range(n_vals):
                pltpu.make_async_copy(adj_hbm.at[pl.ds(0, K)],
                                      small_v.at[_W], sem_v).wait()

        @pl.loop(0, NPAIR)
        def _(t):
            a = 2 * t
            @pl.when(t > 0)
            def _():
                drain(2, 2)
            ha_ = issue(a, 0)
            hb_ = issue(a + 1, 1)
            finish(a, 0, *ha_)
            finish(a + 1, 1, *hb_)

        drain(2, 2)
        tail = NCHUNK - 1
        ht_ = issue(tail, 0)
        finish(tail, 0, *ht_)
        drain(1, 1)

        plsc.subcore_barrier()
        pltpu.sync_copy(acc_sh.at[pl.ds(sid * RPT, RPT)],
                        acc_hbm.at[cid, pl.ds(sid * RPT, RPT)])

    return k(h1, spack, pk3, adj, zeros)


def _sc_prop2(h2, pk3, vals, zeros):
    mesh = plsc.VectorSubcoreMesh(core_axis_name="c", subcore_axis_name="s")

    @functools.partial(
        pl.kernel,
        out_type=jax.ShapeDtypeStruct((NC, NP, D), jnp.float32),
        mesh=mesh,
        compiler_params=pltpu.CompilerParams(needs_layout_passes=False),
        scratch_types=[
            pltpu.VMEM((NCHUNK, K), jnp.int32),
            pltpu.VMEM((K,), jnp.int32),
            pltpu.VMEM((K,), jnp.int32),
            pltpu.VMEM((K,), jnp.int32),
            pltpu.VMEM((K,), jnp.int32),
            pltpu.VMEM((8, K), jnp.float32),
            pltpu.VMEM((K, D), jnp.float32),
            pltpu.VMEM((K, D), jnp.float32),
            pltpu.VMEM_SHARED((NP, D), jnp.float32),
            pltpu.SemaphoreType.DMA,
            pltpu.SemaphoreType.DMA,
            pltpu.SemaphoreType.DMA,
            pltpu.SemaphoreType.DMA,
            pltpu.SemaphoreType.DMA,
        ],
    )
    def k(h2_hbm, pk_hbm, vals_hbm, z_hbm,
          acc_hbm,
          pk2_v, ir0_v, ir1_v, ic0_v, ic1_v, small_v, rows0_v, rows1_v,
          acc_sh, sem_t0, sem_t1, sem_g0, sem_g1, sem_s):
        cid = lax.axis_index("c")
        sid = lax.axis_index("s")
        wid = cid * NS + sid
        pltpu.sync_copy(z_hbm.at[pl.ds(sid * RPT, RPT)],
                        acc_sh.at[pl.ds(sid * RPT, RPT)])
        pltpu.sync_copy(pk_hbm.at[wid], pk2_v)
        plsc.subcore_barrier()

        rows = (rows0_v, rows1_v)
        irb = (ir0_v, ir1_v)
        icb = (ic0_v, ic1_v)
        sem_t = (sem_t0, sem_t1)
        sem_g = (sem_g0, sem_g1)

        def issue(c, buf):
            base = wid * EW + c * K
            _unpack(pk2_v, c, irb[buf], icb[buf])
            hg = pltpu.async_copy(h2_hbm.at[icb[buf]], rows[buf], sem_g[buf])
            hw = pltpu.async_copy(vals_hbm.at[pl.ds(base, K)],
                                  small_v.at[_W + buf], sem_t[buf])
            return hg, hw

        def finish(buf, hg, hw):
            hw.wait()
            hg.wait()
            _scale_rows(rows[buf], small_v, _W + buf)
            pltpu.async_copy(rows[buf], acc_sh.at[irb[buf]], sem_s, add=True)

        def drain(n_scatter):
            for _ in range(n_scatter):
                pltpu.make_async_copy(h2_hbm.at[pl.ds(0, K)], rows0_v,
                                      sem_s).wait()

        @pl.loop(0, NPAIR)
        def _(t):
            a = 2 * t
            @pl.when(t > 0)
            def _():
                drain(2)
            hga, hwa = issue(a, 0)
            hgb, hwb = issue(a + 1, 1)
            finish(0, hga, hwa)
            finish(1, hgb, hwb)

        drain(2)
        tail = NCHUNK - 1
        hgt, hwt = issue(tail, 0)
        finish(0, hgt, hwt)
        drain(1)

        plsc.subcore_barrier()
        pltpu.sync_copy(acc_sh.at[pl.ds(sid * RPT, RPT)],
                        acc_hbm.at[cid, pl.ds(sid * RPT, RPT)])

    return k(h2, pk3, vals, zeros)


def kernel(x, edge_index, adj_values, W1, a1, W2, ln_gamma, ln_beta):
    row = edge_index[0].astype(jnp.int32)
    col = edge_index[1].astype(jnp.int32)
    pk3 = (row * 16384 + col).reshape(NW, NCHUNK, K)
    a_top = jnp.reshape(a1[:D, 0], (1, D))
    a_bot = jnp.reshape(a1[D:, 0], (1, D))
    gamma = jnp.reshape(ln_gamma, (1, D))
    beta = jnp.reshape(ln_beta, (1, D))

    h1, sr, sc = _tc1(x, W1, a_top, a_bot)
    sru = jax.lax.bitcast_convert_type(
        jnp.reshape(sr, (N,)).astype(jnp.bfloat16), jnp.uint16
    ).astype(jnp.uint32)
    scu = jax.lax.bitcast_convert_type(
        jnp.reshape(sc, (N,)).astype(jnp.bfloat16), jnp.uint16
    ).astype(jnp.uint32)
    spack = jax.lax.bitcast_convert_type((sru << 16) | scu, jnp.int32)

    zeros = jnp.zeros((NP, D), jnp.float32)
    acc1, vals = _sc_prop1(h1, spack, pk3, adj_values, zeros)
    h2 = _tc2(acc1, W2)
    acc2 = _sc_prop2(h2, pk3, vals, zeros)
    return _tc3(acc2, x, gamma, beta)


# speedup vs baseline: 11.2221x; 1.1271x over previous
import functools

import jax
import jax.numpy as jnp
from jax import lax
from jax.experimental import pallas as pl
from jax.experimental.pallas import tpu as pltpu
from jax.experimental.pallas import tpu_sc as plsc

N = 10000
D = 128
E = 320000
NC = 2
NS = 16
NW = NC * NS
EW = E // NW
K = 80
NCHUNK = EW // K
NP = 10240
RPT = NP // NS
BLK = 512
GRID = (N + BLK - 1) // BLK


def _tc1_body(x_ref, w1_ref, at_ref, ab_ref, h1_ref, sr_ref, sc_ref):
    h1 = jnp.dot(x_ref[...], w1_ref[...], preferred_element_type=jnp.float32)
    h1_ref[...] = h1
    sr_ref[...] = jnp.sum(h1 * at_ref[...], axis=1).reshape(1, BLK)
    sc_ref[...] = jnp.sum(h1 * ab_ref[...], axis=1).reshape(1, BLK)


def _tc1(x, W1, a_top, a_bot):
    return pl.pallas_call(
        _tc1_body,
        grid=(GRID,),
        in_specs=[
            pl.BlockSpec((BLK, D), lambda i: (i, 0)),
            pl.BlockSpec((D, D), lambda i: (0, 0)),
            pl.BlockSpec((1, D), lambda i: (0, 0)),
            pl.BlockSpec((1, D), lambda i: (0, 0)),
        ],
        out_specs=[
            pl.BlockSpec((BLK, D), lambda i: (i, 0)),
            pl.BlockSpec((1, BLK), lambda i: (0, i)),
            pl.BlockSpec((1, BLK), lambda i: (0, i)),
        ],
        out_shape=[
            jax.ShapeDtypeStruct((N, D), jnp.float32),
            jax.ShapeDtypeStruct((1, N), jnp.float32),
            jax.ShapeDtypeStruct((1, N), jnp.float32),
        ],
    )(x, W1, a_top, a_bot)


def _tc2_body(acc_ref, w2_ref, h2_ref):
    h = jnp.maximum(acc_ref[0] + acc_ref[1], 0.0)
    h2_ref[...] = jnp.dot(h, w2_ref[...], preferred_element_type=jnp.float32)


def _tc2(acc, W2):
    return pl.pallas_call(
        _tc2_body,
        grid=(GRID,),
        in_specs=[
            pl.BlockSpec((NC, BLK, D), lambda i: (0, i, 0)),
            pl.BlockSpec((D, D), lambda i: (0, 0)),
        ],
        out_specs=pl.BlockSpec((BLK, D), lambda i: (i, 0)),
        out_shape=jax.ShapeDtypeStruct((N, D), jnp.float32),
    )(acc, W2)


def _tc3_body(acc_ref, x_ref, g_ref, b_ref, o_ref):
    h = jnp.maximum(acc_ref[0] + acc_ref[1], 0.0) + x_ref[...]
    mu = jnp.mean(h, axis=1, keepdims=True)
    d = h - mu
    var = jnp.mean(d * d, axis=1, keepdims=True)
    o_ref[...] = d * lax.rsqrt(var + 1e-5) * g_ref[...] + b_ref[...]


def _tc3(acc, x, gamma, beta):
    return pl.pallas_call(
        _tc3_body,
        grid=(GRID,),
        in_specs=[
            pl.BlockSpec((NC, BLK, D), lambda i: (0, i, 0)),
            pl.BlockSpec((BLK, D), lambda i: (i, 0)),
            pl.BlockSpec((1, D), lambda i: (0, 0)),
            pl.BlockSpec((1, D), lambda i: (0, 0)),
        ],
        out_specs=pl.BlockSpec((BLK, D), lambda i: (i, 0)),
        out_shape=jax.ShapeDtypeStruct((N, D), jnp.float32),
    )(acc, x, gamma, beta)


_SR, _SC, _W, _AJ = 0, 2, 4, 6


def _unpack_idx(pk_v, ir_v, ic_v):
    for j in range(K // 16):
        sl = pl.ds(16 * j, 16)
        p = pk_v[sl]
        ir_v[sl] = lax.shift_right_logical(p, 14)
        ic_v[sl] = jnp.bitwise_and(p, 16383)


def _weights(small_v, m, with_adj):
    for j in range(K // 16):
        sl = pl.ds(16 * j, 16)
        s = small_v[_SR + m, sl] + small_v[_SC + m, sl]
        s = jnp.where(s >= 0.0, s, 0.2 * s)
        w = 1.0 / (1.0 + jnp.exp(-s))
        if with_adj:
            w = w * small_v[_AJ + m, sl]
        small_v[_W + m, sl] = w


def _scale_rows(gat_v, stag_v, small_v, wrow):
    @pl.loop(0, K // 16)
    def _(g):
        w16 = small_v[wrow, pl.ds(16 * g, 16)]
        for e in range(16):
            wb = jnp.broadcast_to(w16[e], (16,))
            j = 16 * g + e
            for q in range(D // 16):
                sl = pl.ds(16 * q, 16)
                stag_v[j, sl] = gat_v[j, sl] * wb


def _prop_pipeline(first_pass, h_hbm, srow_hbm, scol_hbm, pk_hbm, aux_hbm,
                   z_hbm, vals_hbm, pkb, irb, icb, irs, small_v, gat, stag,
                   acc_sh, sem_p, sem_m, sem_g, sem_v, sem_s, wid):
    def s0_pk(c, m):
        base = wid * EW + c * K
        pltpu.async_copy(pk_hbm.at[pl.ds(base, K)], pkb[m], sem_p[m])

    def s0_aux(c, m):
        base = wid * EW + c * K
        pltpu.async_copy(aux_hbm.at[pl.ds(base, K)],
                         small_v.at[_AJ + m], sem_m[m])

    def s1(c, m):
        g = m % 2
        pltpu.make_async_copy(pk_hbm.at[pl.ds(0, K)], pkb[m], sem_p[m]).wait()
        _unpack_idx(pkb[m], irb[m], icb[m])
        pltpu.async_copy(h_hbm.at[icb[m]], gat[g], sem_g[g])
        if first_pass:
            pltpu.async_copy(srow_hbm.at[irb[m]],
                             small_v.at[_SR + m], sem_m[m])
            pltpu.async_copy(scol_hbm.at[icb[m]],
                             small_v.at[_SC + m], sem_m[m])

    def s2(c, m):
        g = m % 2
        base = wid * EW + c * K

        def _drain_vals():
            pltpu.make_async_copy(aux_hbm.at[pl.ds(0, K)],
                                  small_v.at[_W], sem_v).wait()
        if first_pass:
            if isinstance(c, int):
                if c >= 2:
                    _drain_vals()
            else:
                pl.when(c >= 2)(_drain_vals)
        for _ in range(3 if first_pass else 1):
            pltpu.make_async_copy(aux_hbm.at[pl.ds(0, K)],
                                  small_v.at[_AJ + m], sem_m[m]).wait()
        if first_pass:
            _weights(small_v, m, True)
            pltpu.async_copy(small_v.at[_W + m],
                             vals_hbm.at[pl.ds(base, K)], sem_v)
            wrow = _W + m
        else:
            wrow = _AJ + m
        pltpu.make_async_copy(h_hbm.at[pl.ds(0, K)], gat[g], sem_g[g]).wait()

        def _drain_stag():
            pltpu.make_async_copy(z_hbm.at[pl.ds(0, K)], stag[g],
                                  sem_s).wait()
        if isinstance(c, int):
            if c >= 1:
                _drain_stag()
        else:
            pl.when(c >= 1)(_drain_stag)
        for j in range(K // 16):
            sl = pl.ds(16 * j, 16)
            irs[g][sl] = irb[m][sl]
        _scale_rows(gat[g], stag[g], small_v, wrow)
        pltpu.async_copy(stag[g], acc_sh.at[irs[g]], sem_s, add=True)

    s0_pk(0, 0)
    s0_pk(1, 1)
    s0_aux(0, 0)
    s0_aux(1, 1)
    s1(0, 0)
    s0_pk(2, 0)

    npair = (NCHUNK - 1) // 2
    @pl.loop(0, npair)
    def _(t2):
        for dm in range(2):
            c = 2 * t2 + dm
            mo = (dm + 1) % 2
            s1(c + 1, mo)
            s2(c, dm)

            @pl.when(c + 3 < NCHUNK)
            def _():
                s0_pk(c + 3, mo)

            @pl.when(c + 2 < NCHUNK)
            def _():
                s0_aux(c + 2, dm)

    for c in range(2 * npair, NCHUNK):
        s2(c, c % 2)

    pltpu.make_async_copy(z_hbm.at[pl.ds(0, K)], stag[0], sem_s).wait()
    if first_pass:
        for _ in range(2):
            pltpu.make_async_copy(aux_hbm.at[pl.ds(0, K)],
                                  small_v.at[_W], sem_v).wait()


_SC1_SCRATCH = [
    pltpu.VMEM((K,), jnp.int32),
    pltpu.VMEM((K,), jnp.int32),
    pltpu.VMEM((K,), jnp.int32),
    pltpu.VMEM((K,), jnp.int32),
    pltpu.VMEM((K,), jnp.int32),
    pltpu.VMEM((K,), jnp.int32),
    pltpu.VMEM((K,), jnp.int32),
    pltpu.VMEM((K,), jnp.int32),
    pltpu.VMEM((8, K), jnp.float32),
    pltpu.VMEM((K, D), jnp.float32),
    pltpu.VMEM((K, D), jnp.float32),
    pltpu.VMEM((K, D), jnp.float32),
    pltpu.VMEM((K, D), jnp.float32),
    pltpu.VMEM_SHARED((NP, D), jnp.float32),
    pltpu.SemaphoreType.DMA,
    pltpu.SemaphoreType.DMA,
    pltpu.SemaphoreType.DMA,
    pltpu.SemaphoreType.DMA,
    pltpu.SemaphoreType.DMA,
    pltpu.SemaphoreType.DMA,
    pltpu.SemaphoreType.DMA,
    pltpu.SemaphoreType.DMA,
]


def _sc_prop1(h1, srow, scol, pk, adj, zeros):
    mesh = plsc.VectorSubcoreMesh(core_axis_name="c", subcore_axis_name="s")

    @functools.partial(
        pl.kernel,
        out_type=[
            jax.ShapeDtypeStruct((NC, NP, D), jnp.float32),
            jax.ShapeDtypeStruct((E,), jnp.float32),
        ],
        mesh=mesh,
        compiler_params=pltpu.CompilerParams(needs_layout_passes=False),
        scratch_types=list(_SC1_SCRATCH),
    )
    def k(h1_hbm, srow_hbm, scol_hbm, pk_hbm, adj_hbm, z_hbm,
          acc_hbm, vals_hbm,
          pk0, pk1, ir0, ir1, ic0, ic1, irs0, irs1,
          small_v, gat0, gat1, stag0, stag1, acc_sh,
          sp0, sp1, sm0, sm1, sg0, sg1, sem_v, sem_s):
        cid = lax.axis_index("c")
        sid = lax.axis_index("s")
        wid = cid * NS + sid
        pltpu.sync_copy(z_hbm.at[pl.ds(sid * RPT, RPT)],
                        acc_sh.at[pl.ds(sid * RPT, RPT)])
        plsc.subcore_barrier()

        _prop_pipeline(True, h1_hbm, srow_hbm, scol_hbm, pk_hbm, adj_hbm,
                       z_hbm, vals_hbm,
                       (pk0, pk1), (ir0, ir1),
                       (ic0, ic1), (irs0, irs1), small_v, (gat0, gat1),
                       (stag0, stag1), acc_sh,
                       (sp0, sp1), (sm0, sm1),
                       (sg0, sg1), sem_v, sem_s, wid)

        plsc.subcore_barrier()
        pltpu.sync_copy(acc_sh.at[pl.ds(sid * RPT, RPT)],
                        acc_hbm.at[cid, pl.ds(sid * RPT, RPT)])

    return k(h1, srow, scol, pk, adj, zeros)


def _sc_prop2(h2, pk, vals, zeros):
    mesh = plsc.VectorSubcoreMesh(core_axis_name="c", subcore_axis_name="s")

    @functools.partial(
        pl.kernel,
        out_type=jax.ShapeDtypeStruct((NC, NP, D), jnp.float32),
        mesh=mesh,
        compiler_params=pltpu.CompilerParams(needs_layout_passes=False),
        scratch_types=list(_SC1_SCRATCH),
    )
    def k(h2_hbm, pk_hbm, vals_hbm, z_hbm,
          acc_hbm,
          pk0, pk1, ir0, ir1, ic0, ic1, irs0, irs1,
          small_v, gat0, gat1, stag0, stag1, acc_sh,
          sp0, sp1, sm0, sm1, sg0, sg1, sem_v, sem_s):
        cid = lax.axis_index("c")
        sid = lax.axis_index("s")
        wid = cid * NS + sid
        pltpu.sync_copy(z_hbm.at[pl.ds(sid * RPT, RPT)],
                        acc_sh.at[pl.ds(sid * RPT, RPT)])
        plsc.subcore_barrier()

        _prop_pipeline(False, h2_hbm, None, None, pk_hbm, vals_hbm,
                       z_hbm, None,
                       (pk0, pk1), (ir0, ir1),
                       (ic0, ic1), (irs0, irs1), small_v, (gat0, gat1),
                       (stag0, stag1), acc_sh,
                       (sp0, sp1), (sm0, sm1),
                       (sg0, sg1), sem_v, sem_s, wid)

        plsc.subcore_barrier()
        pltpu.sync_copy(acc_sh.at[pl.ds(sid * RPT, RPT)],
                        acc_hbm.at[cid, pl.ds(sid * RPT, RPT)])

    return k(h2, pk, vals, zeros)


def kernel(x, edge_index, adj_values, W1, a1, W2, ln_gamma, ln_beta):
    row = edge_index[0].astype(jnp.int32)
    col = edge_index[1].astype(jnp.int32)
    pk = row * 16384 + col
    a_top = jnp.reshape(a1[:D, 0], (1, D))
    a_bot = jnp.reshape(a1[D:, 0], (1, D))
    gamma = jnp.reshape(ln_gamma, (1, D))
    beta = jnp.reshape(ln_beta, (1, D))

    h1, sr, sc = _tc1(x, W1, a_top, a_bot)
    srow = jnp.reshape(sr, (N,))
    scol = jnp.reshape(sc, (N,))

    zeros = jnp.zeros((NP, D), jnp.float32)
    acc1, vals = _sc_prop1(h1, srow, scol, pk, adj_values, zeros)
    h2 = _tc2(acc1, W2)
    acc2 = _sc_prop2(h2, pk, vals, zeros)
    return _tc3(acc2, x, gamma, beta)
